# baseline (speedup 1.0000x reference)
import functools
import jax
import jax.numpy as jnp
from jax import lax
from jax.experimental import pallas as pl
from jax.experimental.pallas import tpu as pltpu
from jax.experimental.pallas import tpu_sc as plsc

_N_HASHES = 4
_CHANNELS = 64
_REDUCTION = 4
_CHUNK = 144
_CR = _CHANNELS // _REDUCTION
_ROW = 256
_OROW = 128

_NW = 32
_SCCHUNK = 128


def _embed_body(x_ref, y_ref, f_ref, fc1_w, fc1_b, fc2_w, rot_ref, out_ref, codes_ref):
    BLK = x_ref.shape[0]
    h1 = jax.nn.relu(
        lax.dot_general(f_ref[...], fc1_w[...], (((1,), (1,)), ((), ())),
                        preferred_element_type=jnp.float32) + fc1_b[...])
    t = lax.dot_general(h1, fc2_w[...], (((1,), (1,)), ((), ())),
                        preferred_element_type=jnp.float32)
    pad = jnp.zeros((BLK, _ROW - _CR - _CHANNELS - _CHUNK), jnp.float32)
    out_ref[...] = jnp.concatenate([x_ref[...], y_ref[...], t, pad], axis=1)

    r = lax.dot_general(x_ref[...], rot_ref[...], (((1,), (0,)), ((), ())),
                        preferred_element_type=jnp.float32)
    cols = []
    iota = lax.broadcasted_iota(jnp.int32, (BLK, 64), 1)
    for h in range(_N_HASHES):
        sub = r[:, h * 64:(h + 1) * 64]
        m = jnp.max(sub, axis=1, keepdims=True)
        idx = jnp.min(jnp.where(sub == m, iota, 64), axis=1, keepdims=True)
        cols.append(idx + h * 64)
    codes_ref[...] = jnp.concatenate(cols, axis=1)


def _build_table(x_embed, y_embed, f_embed, fc1_w, fc1_b, fc2_w, rot):
    NL = x_embed.shape[0]
    BLK = 1024
    grid = (NL // BLK,)
    return pl.pallas_call(
        _embed_body,
        grid=grid,
        in_specs=[
            pl.BlockSpec((BLK, _CR), lambda i: (i, 0)),
            pl.BlockSpec((BLK, _CHANNELS), lambda i: (i, 0)),
            pl.BlockSpec((BLK, _CHANNELS), lambda i: (i, 0)),
            pl.BlockSpec((_CHUNK, _CHANNELS), lambda i: (0, 0)),
            pl.BlockSpec((1, _CHUNK), lambda i: (0, 0)),
            pl.BlockSpec((_CHUNK, _CHUNK), lambda i: (0, 0)),
            pl.BlockSpec((_CR, _N_HASHES * 64), lambda i: (0, 0)),
        ],
        out_specs=[
            pl.BlockSpec((BLK, _ROW), lambda i: (i, 0)),
            pl.BlockSpec((BLK, _N_HASHES), lambda i: (i, 0)),
        ],
        out_shape=[
            jax.ShapeDtypeStruct((NL, _ROW), jnp.float32),
            jax.ShapeDtypeStruct((NL, _N_HASHES), jnp.int32),
        ],
    )(x_embed, y_embed, f_embed, fc1_w, fc1_b.reshape(1, -1), fc2_w, rot)


def _sc_gather(table, gidx3d, D):
    B = _NW * gidx3d.shape[1] * _SCCHUNK
    per_w = B // _NW
    n_ch = per_w // _SCCHUNK
    mesh = plsc.VectorSubcoreMesh(core_axis_name="c", subcore_axis_name="s")

    @functools.partial(
        pl.kernel, mesh=mesh,
        out_type=jax.ShapeDtypeStruct((B, D), jnp.float32),
        scratch_types=[
            pltpu.VMEM((n_ch, _SCCHUNK), jnp.int32),
            pltpu.VMEM((_SCCHUNK, D), jnp.float32),
            pltpu.SemaphoreType.DMA,
        ],
    )
    def k(table_hbm, idx_hbm, out_hbm, idx_v, rows_v, sem):
        wid = lax.axis_index("s") * 2 + lax.axis_index("c")
        pltpu.sync_copy(idx_hbm.at[wid], idx_v)

        def body(j, _):
            pltpu.async_copy(table_hbm.at[idx_v.at[j]], rows_v, sem).wait()
            base = wid * per_w + j * _SCCHUNK
            pltpu.sync_copy(rows_v, out_hbm.at[pl.ds(base, _SCCHUNK)])
            return 0

        lax.fori_loop(0, n_ch, body, 0)

    return k(table, gidx3d)


def _sc_scatter(rows, gdst3d, D):
    B = _NW * gdst3d.shape[1] * _SCCHUNK
    per_w = B // _NW
    n_ch = per_w // _SCCHUNK
    mesh = plsc.VectorSubcoreMesh(core_axis_name="c", subcore_axis_name="s")

    @functools.partial(
        pl.kernel, mesh=mesh,
        out_type=jax.ShapeDtypeStruct((B, D), jnp.float32),
        scratch_types=[
            pltpu.VMEM((n_ch, _SCCHUNK), jnp.int32),
            pltpu.VMEM((_SCCHUNK, D), jnp.float32),
            pltpu.SemaphoreType.DMA,
        ],
    )
    def k(rows_hbm, idx_hbm, out_hbm, idx_v, rows_v, sem):
        wid = lax.axis_index("s") * 2 + lax.axis_index("c")
        pltpu.sync_copy(idx_hbm.at[wid], idx_v)

        def body(j, _):
            base = wid * per_w + j * _SCCHUNK
            pltpu.sync_copy(rows_hbm.at[pl.ds(base, _SCCHUNK)], rows_v)
            pltpu.async_copy(rows_v, out_hbm.at[idx_v.at[j]], sem).wait()
            return 0

        lax.fori_loop(0, n_ch, body, 0)

    return k(rows, gdst3d)


_NBINS = 256
_NSUB = 16


def _sc_counting_sort(codes, NB, Ltot):
    per_w = Ltot // _NSUB
    nvec = per_w // 16
    mesh = plsc.VectorSubcoreMesh(core_axis_name="c", subcore_axis_name="s")

    @functools.partial(
        pl.kernel, mesh=mesh,
        out_type=jax.ShapeDtypeStruct((NB, Ltot), jnp.int32),
        compiler_params=pltpu.CompilerParams(needs_layout_passes=False),
        scratch_types=[
            pltpu.VMEM((per_w,), jnp.int32),
            pltpu.VMEM((16 * _NBINS,), jnp.int32),
            pltpu.VMEM((_NBINS,), jnp.int32),
            pltpu.VMEM((_NSUB, _NBINS), jnp.int32),
            pltpu.VMEM((_NBINS,), jnp.int32),
            pltpu.VMEM((per_w,), jnp.int32),
            pltpu.VMEM((per_w,), jnp.int32),
            pltpu.VMEM_SHARED((_NSUB, _NBINS), jnp.int32),
            pltpu.VMEM_SHARED((Ltot,), jnp.int32),
        ],
    )
    def k(codes_hbm, out_hbm, keys_v, hist_v, histred_v, allhist_v, offs_v,
          dst_v, pos_v, sh_hist, sh_out):
        b = lax.axis_index("c")
        s = lax.axis_index("s")
        base = s * per_w
        lane = lax.iota(jnp.int32, 16)
        ones16 = jnp.ones((16,), jnp.int32)

        pltpu.sync_copy(codes_hbm.at[b, pl.ds(base, per_w)], keys_v)

        def zero_body(i, _):
            hist_v[pl.ds(i * 16, 16)] = jnp.zeros((16,), jnp.int32)
            return 0
        lax.fori_loop(0, 16 * _NBINS // 16, zero_body, 0)

        def hist_body(i, _):
            k16 = keys_v[pl.ds(i * 16, 16)]
            plsc.addupdate_scatter(hist_v, [lane * _NBINS + k16], ones16)
            return 0
        lax.fori_loop(0, nvec, hist_body, 0)

        for j in range(_NBINS // 16):
            acc = jnp.zeros((16,), jnp.int32)
            for l in range(16):
                acc = acc + hist_v[pl.ds(l * _NBINS + j * 16, 16)]
            histred_v[pl.ds(j * 16, 16)] = acc

        pltpu.sync_copy(histred_v, sh_hist.at[s])
        plsc.subcore_barrier()
        pltpu.sync_copy(sh_hist, allhist_v)

        carry = jnp.int32(0)
        for j in range(_NBINS // 16):
            tot = jnp.zeros((16,), jnp.int32)
            mine = jnp.zeros((16,), jnp.int32)
            for l in range(_NSUB):
                row = allhist_v[l, pl.ds(j * 16, 16)]
                tot = tot + row
                mine = mine + row * jnp.where(l < s, jnp.int32(1), jnp.int32(0))
            inc = plsc.cumsum(tot)
            offs_v[pl.ds(j * 16, 16)] = (inc - tot) + mine + carry
            carry = carry + jnp.sum(tot)

        def rank_body(i, _):
            k16 = keys_v[pl.ds(i * 16, 16)]
            dst16 = jnp.zeros((16,), jnp.int32)
            for l in range(16):
                m = lane == l
                d = plsc.load_gather(offs_v, [k16])
                dst16 = jnp.where(m, d, dst16)
                plsc.addupdate_scatter(offs_v, [k16], ones16, mask=m)
            dst_v[pl.ds(i * 16, 16)] = dst16
            pos_v[pl.ds(i * 16, 16)] = base + i * 16 + lane
            return 0
        lax.fori_loop(0, nvec, rank_body, 0)

        pltpu.sync_copy(pos_v, sh_out.at[dst_v])
        plsc.subcore_barrier()
        pltpu.sync_copy(sh_out.at[pl.ds(base, per_w)], out_hbm.at[b, pl.ds(base, per_w)])

    return k(codes)


def _attn_body(rows_ref, fc2_b, out_ref):
    K = rows_ref.shape[1] // _CHUNK
    eye = jnp.eye(_CHUNK, dtype=jnp.float32)
    zpad = jnp.zeros((_CHUNK, _OROW - _CHANNELS - 1), dtype=jnp.float32)

    def chunk(start):
        return rows_ref[0, pl.ds(start, _CHUNK), :]

    t0 = _CR + _CHANNELS

    def body(k, _):
        cur = chunk(k * _CHUNK)
        prv = chunk(lax.rem(k + K - 1, K) * _CHUNK)
        nxt = chunk(lax.rem(k + 1, K) * _CHUNK)
        xq = cur[:, :_CR]

        def raw_of(t):
            x = t[:, :_CR]
            n = jnp.sqrt(jnp.sum(x * x, axis=-1, keepdims=True))
            xn = x / jnp.maximum(n, 5e-05)
            qk = lax.dot_general(xn, xq, (((1,), (1,)), ((), ())),
                                 preferred_element_type=jnp.float32)
            return qk + t[:, t0:t0 + _CHUNK] + fc2_b[...]

        raws = [raw_of(cur), raw_of(prv), raw_of(nxt)]
        m = jnp.maximum(jnp.maximum(jnp.max(raws[0], axis=0, keepdims=True),
                                    jnp.max(raws[1], axis=0, keepdims=True)),
                        jnp.max(raws[2], axis=0, keepdims=True))
        es = [jnp.exp(r - m) for r in raws]
        s = (jnp.sum(es[0], axis=0, keepdims=True)
             + jnp.sum(es[1], axis=0, keepdims=True)
             + jnp.sum(es[2], axis=0, keepdims=True))
        blocks = [cur, prv, nxt]
        ret = sum(
            lax.dot_general(es[p], blocks[p][:, _CR:_CR + _CHANNELS],
                            (((0,), (0,)), ((), ())),
                            preferred_element_type=jnp.float32)
            for p in range(3))
        bsms = m + jnp.log(s)
        bscol = lax.dot_general(eye, bsms, (((1,), (1,)), ((), ())),
                                preferred_element_type=jnp.float32)
        scol = lax.dot_general(eye, s, (((1,), (1,)), ((), ())),
                               preferred_element_type=jnp.float32)
        ret = ret * (1.0 / scol)
        out_ref[0, pl.ds(k * _CHUNK, _CHUNK), :] = jnp.concatenate(
            [ret, bscol, zpad], axis=1)
        return 0

    lax.fori_loop(0, K, body, 0)


def _bucket_attention(rows_sorted, fc2_b, G, LH):
    return pl.pallas_call(
        _attn_body,
        grid=(G,),
        in_specs=[
            pl.BlockSpec((1, LH, _ROW), lambda h: (h, 0, 0)),
            pl.BlockSpec((1, _CHUNK), lambda h: (0, 0)),
        ],
        out_specs=pl.BlockSpec((1, LH, _OROW), lambda h: (h, 0, 0)),
        out_shape=jax.ShapeDtypeStruct((G, LH, _OROW), jnp.float32),
    )(rows_sorted, fc2_b.reshape(1, -1))


def _conv2d_relu_nhwc(x, w, b):
    y = lax.conv_general_dilated(x, jnp.transpose(w, (2, 3, 1, 0)),
                                 window_strides=(1, 1), padding='SAME',
                                 dimension_numbers=('NHWC', 'HWIO', 'NHWC'))
    return jax.nn.relu(y + b[None, None, None, :])


def kernel(input, cm_w, cm_b, ca_w, ca_b, cf_w, cf_b, fc1_w, fc1_b, fc2_w, fc2_b, random_rotations):
    N, H, W, _ = input.shape
    L = H * W
    x_embed = _conv2d_relu_nhwc(input, cm_w, cm_b).reshape(N * L, -1)
    y_embed = _conv2d_relu_nhwc(input, ca_w, ca_b).reshape(N * L, -1)
    fc_embed = _conv2d_relu_nhwc(input, cf_w, cf_b).reshape(N * L, -1)
    C = x_embed.shape[-1]
    hb = min(L // _CHUNK + (L // _CHUNK) % 2, 128)

    rot = random_rotations.reshape(C, _N_HASHES * hb)
    table, codes = _build_table(x_embed, y_embed, fc_embed,
                                fc1_w, fc1_b, fc2_w, rot)

    hash_codes = codes.reshape(N, L, _N_HASHES).transpose(0, 2, 1).reshape(N, -1)
    indices = _sc_counting_sort(hash_codes.astype(jnp.int32), N, hash_codes.shape[1])

    HL = _N_HASHES * L
    gidx = (indices % L + (jnp.arange(N) * L)[:, None]).astype(jnp.int32)
    rows_sorted = _sc_gather(table, gidx.reshape(_NW, -1, _SCCHUNK), _ROW)

    G = N * _N_HASHES
    LH = L
    out80 = _bucket_attention(rows_sorted.reshape(G, LH, _ROW), fc2_b, G, LH)

    gdst = (indices + (jnp.arange(N) * HL)[:, None]).astype(jnp.int32)
    unsorted = _sc_scatter(out80.reshape(N * HL, _OROW), gdst.reshape(_NW, -1, _SCCHUNK), _OROW)

    ret = unsorted[:, :_CHANNELS].reshape(N, _N_HASHES, L, _CHANNELS)
    bscore = unsorted[:, _CHANNELS].reshape(N, _N_HASHES, L, 1)
    probs = jax.nn.softmax(bscore, axis=1)
    ret = jnp.sum(ret * probs, axis=1).reshape(N, H, W, -1)
    return ret + input


# speedup vs baseline: 1.0439x; 1.0439x over previous
import functools
import jax
import jax.numpy as jnp
from jax import lax
from jax.experimental import pallas as pl
from jax.experimental.pallas import tpu as pltpu
from jax.experimental.pallas import tpu_sc as plsc

_N_HASHES = 4
_CHANNELS = 64
_REDUCTION = 4
_CHUNK = 144
_CR = _CHANNELS // _REDUCTION
_ROW = 256
_OROW = 128

_NW = 32
_SCCHUNK = 128


def _embed_body(x_ref, y_ref, f_ref, fc1_w, fc1_b, fc2_w, rot_ref, out_ref, codes_ref):
    BLK = x_ref.shape[0]
    h1 = jax.nn.relu(
        lax.dot_general(f_ref[...], fc1_w[...], (((1,), (1,)), ((), ())),
                        preferred_element_type=jnp.float32) + fc1_b[...])
    t = lax.dot_general(h1, fc2_w[...], (((1,), (1,)), ((), ())),
                        preferred_element_type=jnp.float32)
    pad = jnp.zeros((BLK, _ROW - _CR - _CHANNELS - _CHUNK), jnp.float32)
    out_ref[...] = jnp.concatenate([x_ref[...], y_ref[...], t, pad], axis=1)

    r = lax.dot_general(x_ref[...], rot_ref[...], (((1,), (0,)), ((), ())),
                        preferred_element_type=jnp.float32)
    cols = []
    iota = lax.broadcasted_iota(jnp.int32, (BLK, 64), 1)
    for h in range(_N_HASHES):
        sub = r[:, h * 64:(h + 1) * 64]
        m = jnp.max(sub, axis=1, keepdims=True)
        idx = jnp.min(jnp.where(sub == m, iota, 64), axis=1, keepdims=True)
        cols.append(idx + h * 64)
    codes_ref[...] = jnp.concatenate(cols, axis=1)


def _build_table(x_embed, y_embed, f_embed, fc1_w, fc1_b, fc2_w, rot):
    NL = x_embed.shape[0]
    BLK = 1024
    grid = (NL // BLK,)
    return pl.pallas_call(
        _embed_body,
        grid=grid,
        in_specs=[
            pl.BlockSpec((BLK, _CR), lambda i: (i, 0)),
            pl.BlockSpec((BLK, _CHANNELS), lambda i: (i, 0)),
            pl.BlockSpec((BLK, _CHANNELS), lambda i: (i, 0)),
            pl.BlockSpec((_CHUNK, _CHANNELS), lambda i: (0, 0)),
            pl.BlockSpec((1, _CHUNK), lambda i: (0, 0)),
            pl.BlockSpec((_CHUNK, _CHUNK), lambda i: (0, 0)),
            pl.BlockSpec((_CR, _N_HASHES * 64), lambda i: (0, 0)),
        ],
        out_specs=[
            pl.BlockSpec((BLK, _ROW), lambda i: (i, 0)),
            pl.BlockSpec((BLK, _N_HASHES), lambda i: (i, 0)),
        ],
        out_shape=[
            jax.ShapeDtypeStruct((NL, _ROW), jnp.float32),
            jax.ShapeDtypeStruct((NL, _N_HASHES), jnp.int32),
        ],
    )(x_embed, y_embed, f_embed, fc1_w, fc1_b.reshape(1, -1), fc2_w, rot)


def _sc_gather(table, gidx3d, D):
    B = _NW * gidx3d.shape[1] * _SCCHUNK
    per_w = B // _NW
    n_ch = per_w // _SCCHUNK
    mesh = plsc.VectorSubcoreMesh(core_axis_name="c", subcore_axis_name="s")

    @functools.partial(
        pl.kernel, mesh=mesh,
        out_type=jax.ShapeDtypeStruct((B, D), jnp.float32),
        scratch_types=[
            pltpu.VMEM((n_ch, _SCCHUNK), jnp.int32),
            pltpu.VMEM((_SCCHUNK, D), jnp.float32),
            pltpu.SemaphoreType.DMA,
        ],
    )
    def k(table_hbm, idx_hbm, out_hbm, idx_v, rows_v, sem):
        wid = lax.axis_index("s") * 2 + lax.axis_index("c")
        pltpu.sync_copy(idx_hbm.at[wid], idx_v)

        def body(j, _):
            pltpu.async_copy(table_hbm.at[idx_v.at[j]], rows_v, sem).wait()
            base = wid * per_w + j * _SCCHUNK
            pltpu.sync_copy(rows_v, out_hbm.at[pl.ds(base, _SCCHUNK)])
            return 0

        lax.fori_loop(0, n_ch, body, 0)

    return k(table, gidx3d)


def _sc_scatter(rows, gdst3d, D):
    B = _NW * gdst3d.shape[1] * _SCCHUNK
    per_w = B // _NW
    n_ch = per_w // _SCCHUNK
    mesh = plsc.VectorSubcoreMesh(core_axis_name="c", subcore_axis_name="s")

    @functools.partial(
        pl.kernel, mesh=mesh,
        out_type=jax.ShapeDtypeStruct((B, D), jnp.float32),
        scratch_types=[
            pltpu.VMEM((n_ch, _SCCHUNK), jnp.int32),
            pltpu.VMEM((_SCCHUNK, D), jnp.float32),
            pltpu.SemaphoreType.DMA,
        ],
    )
    def k(rows_hbm, idx_hbm, out_hbm, idx_v, rows_v, sem):
        wid = lax.axis_index("s") * 2 + lax.axis_index("c")
        pltpu.sync_copy(idx_hbm.at[wid], idx_v)

        def body(j, _):
            base = wid * per_w + j * _SCCHUNK
            pltpu.sync_copy(rows_hbm.at[pl.ds(base, _SCCHUNK)], rows_v)
            pltpu.async_copy(rows_v, out_hbm.at[idx_v.at[j]], sem).wait()
            return 0

        lax.fori_loop(0, n_ch, body, 0)

    return k(rows, gdst3d)


_NBINS = 256
_NSUB = 16


def _sc_counting_sort(codes, NB, Ltot):
    per_w = Ltot // _NSUB
    nvec = per_w // 16
    mesh = plsc.VectorSubcoreMesh(core_axis_name="c", subcore_axis_name="s")

    @functools.partial(
        pl.kernel, mesh=mesh,
        out_type=jax.ShapeDtypeStruct((NB, Ltot), jnp.int32),
        compiler_params=pltpu.CompilerParams(needs_layout_passes=False),
        scratch_types=[
            pltpu.VMEM((per_w,), jnp.int32),
            pltpu.VMEM((16 * _NBINS,), jnp.int32),
            pltpu.VMEM((_NBINS,), jnp.int32),
            pltpu.VMEM((_NSUB, _NBINS), jnp.int32),
            pltpu.VMEM((_NBINS,), jnp.int32),
            pltpu.VMEM((per_w,), jnp.int32),
            pltpu.VMEM((per_w,), jnp.int32),
            pltpu.VMEM_SHARED((_NSUB, _NBINS), jnp.int32),
            pltpu.VMEM_SHARED((Ltot,), jnp.int32),
        ],
    )
    def k(codes_hbm, out_hbm, keys_v, hist_v, histred_v, allhist_v, offs_v,
          dst_v, pos_v, sh_hist, sh_out):
        b = lax.axis_index("c")
        s = lax.axis_index("s")
        base = s * per_w
        lane = lax.iota(jnp.int32, 16)
        ones16 = jnp.ones((16,), jnp.int32)

        pltpu.sync_copy(codes_hbm.at[b, pl.ds(base, per_w)], keys_v)

        def zero_body(i, _):
            hist_v[pl.ds(i * 16, 16)] = jnp.zeros((16,), jnp.int32)
            return 0
        lax.fori_loop(0, 16 * _NBINS // 16, zero_body, 0)

        def hist_body(i, _):
            k16 = keys_v[pl.ds(i * 16, 16)]
            plsc.addupdate_scatter(hist_v, [lane * _NBINS + k16], ones16)
            return 0
        lax.fori_loop(0, nvec, hist_body, 0)

        for j in range(_NBINS // 16):
            acc = jnp.zeros((16,), jnp.int32)
            for l in range(16):
                acc = acc + hist_v[pl.ds(l * _NBINS + j * 16, 16)]
            histred_v[pl.ds(j * 16, 16)] = acc

        pltpu.sync_copy(histred_v, sh_hist.at[s])
        plsc.subcore_barrier()
        pltpu.sync_copy(sh_hist, allhist_v)

        carry = jnp.int32(0)
        for j in range(_NBINS // 16):
            tot = jnp.zeros((16,), jnp.int32)
            mine = jnp.zeros((16,), jnp.int32)
            for l in range(_NSUB):
                row = allhist_v[l, pl.ds(j * 16, 16)]
                tot = tot + row
                mine = mine + row * jnp.where(l < s, jnp.int32(1), jnp.int32(0))
            inc = plsc.cumsum(tot)
            offs_v[pl.ds(j * 16, 16)] = (inc - tot) + mine + carry
            carry = carry + jnp.sum(tot)

        def rank_body(i, _):
            k16 = keys_v[pl.ds(i * 16, 16)]
            dst16 = jnp.zeros((16,), jnp.int32)
            for l in range(16):
                m = lane == l
                d = plsc.load_gather(offs_v, [k16])
                dst16 = jnp.where(m, d, dst16)
                plsc.addupdate_scatter(offs_v, [k16], ones16, mask=m)
            dst_v[pl.ds(i * 16, 16)] = dst16
            pos_v[pl.ds(i * 16, 16)] = base + i * 16 + lane
            return 0
        lax.fori_loop(0, nvec, rank_body, 0)

        pltpu.sync_copy(pos_v, sh_out.at[dst_v])
        plsc.subcore_barrier()
        pltpu.sync_copy(sh_out.at[pl.ds(base, per_w)], out_hbm.at[b, pl.ds(base, per_w)])

    return k(codes)


def _attn_body(rows_ref, fc2_b, out_ref):
    K = rows_ref.shape[1] // _CHUNK
    eye = jnp.eye(_CHUNK, dtype=jnp.float32)
    zpad = jnp.zeros((_CHUNK, _OROW - _CHANNELS - 1), dtype=jnp.float32)

    def chunk(start):
        return rows_ref[0, pl.ds(start, _CHUNK), :]

    t0 = _CR + _CHANNELS

    def body(k, _):
        cur = chunk(k * _CHUNK)
        prv = chunk(lax.rem(k + K - 1, K) * _CHUNK)
        nxt = chunk(lax.rem(k + 1, K) * _CHUNK)
        xq = cur[:, :_CR]

        def nrm(t):
            x = t[:, :_CR]
            n = jnp.sqrt(jnp.sum(x * x, axis=-1, keepdims=True))
            return x / jnp.maximum(n, 5e-05)

        xm = jnp.concatenate([nrm(cur), nrm(prv), nrm(nxt)], axis=0)
        yc = jnp.concatenate([cur[:, _CR:t0],
                              prv[:, _CR:t0],
                              nxt[:, _CR:t0]], axis=0)
        tc = jnp.concatenate([cur[:, t0:t0 + _CHUNK],
                              prv[:, t0:t0 + _CHUNK],
                              nxt[:, t0:t0 + _CHUNK]], axis=0)

        raw_t = lax.dot_general(xm, xq, (((1,), (1,)), ((), ())),
                                preferred_element_type=jnp.float32) + tc + fc2_b[...]
        m = jnp.max(raw_t, axis=0, keepdims=True)
        e = jnp.exp(raw_t - m)
        s = jnp.sum(e, axis=0, keepdims=True)
        ret = lax.dot_general(e, yc, (((0,), (0,)), ((), ())),
                              preferred_element_type=jnp.float32)
        bsms = m + jnp.log(s)
        bscol = lax.dot_general(eye, bsms, (((1,), (1,)), ((), ())),
                                preferred_element_type=jnp.float32)
        scol = lax.dot_general(eye, s, (((1,), (1,)), ((), ())),
                               preferred_element_type=jnp.float32)
        ret = ret * (1.0 / scol)
        out_ref[0, pl.ds(k * _CHUNK, _CHUNK), :] = jnp.concatenate(
            [ret, bscol, zpad], axis=1)
        return 0

    lax.fori_loop(0, K, body, 0)


def _bucket_attention(rows_sorted, fc2_b, G, LH):
    return pl.pallas_call(
        _attn_body,
        grid=(G,),
        in_specs=[
            pl.BlockSpec((1, LH, _ROW), lambda h: (h, 0, 0)),
            pl.BlockSpec((1, _CHUNK), lambda h: (0, 0)),
        ],
        out_specs=pl.BlockSpec((1, LH, _OROW), lambda h: (h, 0, 0)),
        out_shape=jax.ShapeDtypeStruct((G, LH, _OROW), jnp.float32),
    )(rows_sorted, fc2_b.reshape(1, -1))


def _conv2d_relu_nhwc(x, w, b):
    y = lax.conv_general_dilated(x, jnp.transpose(w, (2, 3, 1, 0)),
                                 window_strides=(1, 1), padding='SAME',
                                 dimension_numbers=('NHWC', 'HWIO', 'NHWC'))
    return jax.nn.relu(y + b[None, None, None, :])


def kernel(input, cm_w, cm_b, ca_w, ca_b, cf_w, cf_b, fc1_w, fc1_b, fc2_w, fc2_b, random_rotations):
    N, H, W, _ = input.shape
    L = H * W
    x_embed = _conv2d_relu_nhwc(input, cm_w, cm_b).reshape(N * L, -1)
    y_embed = _conv2d_relu_nhwc(input, ca_w, ca_b).reshape(N * L, -1)
    fc_embed = _conv2d_relu_nhwc(input, cf_w, cf_b).reshape(N * L, -1)
    C = x_embed.shape[-1]
    hb = min(L // _CHUNK + (L // _CHUNK) % 2, 128)

    rot = random_rotations.reshape(C, _N_HASHES * hb)
    table, codes = _build_table(x_embed, y_embed, fc_embed,
                                fc1_w, fc1_b, fc2_w, rot)

    hash_codes = codes.reshape(N, L, _N_HASHES).transpose(0, 2, 1).reshape(N, -1)
    indices = _sc_counting_sort(hash_codes.astype(jnp.int32), N, hash_codes.shape[1])

    HL = _N_HASHES * L
    gidx = (indices % L + (jnp.arange(N) * L)[:, None]).astype(jnp.int32)
    rows_sorted = _sc_gather(table, gidx.reshape(_NW, -1, _SCCHUNK), _ROW)

    G = N * _N_HASHES
    LH = L
    out80 = _bucket_attention(rows_sorted.reshape(G, LH, _ROW), fc2_b, G, LH)

    gdst = (indices + (jnp.arange(N) * HL)[:, None]).astype(jnp.int32)
    unsorted = _sc_scatter(out80.reshape(N * HL, _OROW), gdst.reshape(_NW, -1, _SCCHUNK), _OROW)

    ret = unsorted[:, :_CHANNELS].reshape(N, _N_HASHES, L, _CHANNELS)
    bscore = unsorted[:, _CHANNELS].reshape(N, _N_HASHES, L, 1)
    probs = jax.nn.softmax(bscore, axis=1)
    ret = jnp.sum(ret * probs, axis=1).reshape(N, H, W, -1)
    return ret + input


# speedup vs baseline: 1.1875x; 1.1376x over previous
import functools
import jax
import jax.numpy as jnp
from jax import lax
from jax.experimental import pallas as pl
from jax.experimental.pallas import tpu as pltpu
from jax.experimental.pallas import tpu_sc as plsc

_N_HASHES = 4
_CHANNELS = 64
_REDUCTION = 4
_CHUNK = 144
_CR = _CHANNELS // _REDUCTION
_ROW = 256
_OROW = 128

_NW = 32
_SCCHUNK = 128


def _embed_body(x_ref, y_ref, f_ref, fc1_w, fc1_b, fc2_w, out_ref):
    BLK = x_ref.shape[0]
    h1 = jax.nn.relu(
        lax.dot_general(f_ref[...], fc1_w[...], (((1,), (1,)), ((), ())),
                        preferred_element_type=jnp.float32) + fc1_b[...])
    t = lax.dot_general(h1, fc2_w[...], (((1,), (1,)), ((), ())),
                        preferred_element_type=jnp.float32)
    pad = jnp.zeros((BLK, _ROW - _CR - _CHANNELS - _CHUNK), jnp.float32)
    out_ref[...] = jnp.concatenate([x_ref[...], y_ref[...], t, pad], axis=1)


def _build_table(x_embed, y_embed, f_embed, fc1_w, fc1_b, fc2_w):
    NL = x_embed.shape[0]
    BLK = 1024
    grid = (NL // BLK,)
    return pl.pallas_call(
        _embed_body,
        grid=grid,
        in_specs=[
            pl.BlockSpec((BLK, _CR), lambda i: (i, 0)),
            pl.BlockSpec((BLK, _CHANNELS), lambda i: (i, 0)),
            pl.BlockSpec((BLK, _CHANNELS), lambda i: (i, 0)),
            pl.BlockSpec((_CHUNK, _CHANNELS), lambda i: (0, 0)),
            pl.BlockSpec((1, _CHUNK), lambda i: (0, 0)),
            pl.BlockSpec((_CHUNK, _CHUNK), lambda i: (0, 0)),
        ],
        out_specs=pl.BlockSpec((BLK, _ROW), lambda i: (i, 0)),
        out_shape=jax.ShapeDtypeStruct((NL, _ROW), jnp.float32),
    )(x_embed, y_embed, f_embed, fc1_w, fc1_b.reshape(1, -1), fc2_w)


def _sc_gather(table, gidx3d, D):
    B = _NW * gidx3d.shape[1] * _SCCHUNK
    per_w = B // _NW
    n_ch = per_w // _SCCHUNK
    mesh = plsc.VectorSubcoreMesh(core_axis_name="c", subcore_axis_name="s")

    @functools.partial(
        pl.kernel, mesh=mesh,
        out_type=jax.ShapeDtypeStruct((B, D), jnp.float32),
        scratch_types=[
            pltpu.VMEM((n_ch, _SCCHUNK), jnp.int32),
            pltpu.VMEM((_SCCHUNK, D), jnp.float32),
            pltpu.SemaphoreType.DMA,
        ],
    )
    def k(table_hbm, idx_hbm, out_hbm, idx_v, rows_v, sem):
        wid = lax.axis_index("s") * 2 + lax.axis_index("c")
        pltpu.sync_copy(idx_hbm.at[wid], idx_v)

        def body(j, _):
            pltpu.async_copy(table_hbm.at[idx_v.at[j]], rows_v, sem).wait()
            base = wid * per_w + j * _SCCHUNK
            pltpu.sync_copy(rows_v, out_hbm.at[pl.ds(base, _SCCHUNK)])
            return 0

        lax.fori_loop(0, n_ch, body, 0)

    return k(table, gidx3d)


def _sc_scatter(rows, gdst3d, D):
    B = _NW * gdst3d.shape[1] * _SCCHUNK
    per_w = B // _NW
    n_ch = per_w // _SCCHUNK
    mesh = plsc.VectorSubcoreMesh(core_axis_name="c", subcore_axis_name="s")

    @functools.partial(
        pl.kernel, mesh=mesh,
        out_type=jax.ShapeDtypeStruct((B, D), jnp.float32),
        scratch_types=[
            pltpu.VMEM((n_ch, _SCCHUNK), jnp.int32),
            pltpu.VMEM((_SCCHUNK, D), jnp.float32),
            pltpu.SemaphoreType.DMA,
        ],
    )
    def k(rows_hbm, idx_hbm, out_hbm, idx_v, rows_v, sem):
        wid = lax.axis_index("s") * 2 + lax.axis_index("c")
        pltpu.sync_copy(idx_hbm.at[wid], idx_v)

        def body(j, _):
            base = wid * per_w + j * _SCCHUNK
            pltpu.sync_copy(rows_hbm.at[pl.ds(base, _SCCHUNK)], rows_v)
            pltpu.async_copy(rows_v, out_hbm.at[idx_v.at[j]], sem).wait()
            return 0

        lax.fori_loop(0, n_ch, body, 0)

    return k(rows, gdst3d)


_NBINS = 256
_NSUB = 16


def _sc_counting_sort(codes, NB, Ltot):
    per_w = Ltot // _NSUB
    nvec = per_w // 16
    mesh = plsc.VectorSubcoreMesh(core_axis_name="c", subcore_axis_name="s")

    @functools.partial(
        pl.kernel, mesh=mesh,
        out_type=jax.ShapeDtypeStruct((NB, Ltot), jnp.int32),
        compiler_params=pltpu.CompilerParams(needs_layout_passes=False),
        scratch_types=[
            pltpu.VMEM((per_w,), jnp.int32),
            pltpu.VMEM((16 * _NBINS,), jnp.int32),
            pltpu.VMEM((_NBINS,), jnp.int32),
            pltpu.VMEM((_NSUB, _NBINS), jnp.int32),
            pltpu.VMEM((_NBINS,), jnp.int32),
            pltpu.VMEM((per_w,), jnp.int32),
            pltpu.VMEM((per_w,), jnp.int32),
            pltpu.VMEM_SHARED((_NSUB, _NBINS), jnp.int32),
            pltpu.VMEM_SHARED((Ltot,), jnp.int32),
        ],
    )
    def k(codes_hbm, out_hbm, keys_v, hist_v, histred_v, allhist_v, offs_v,
          dst_v, pos_v, sh_hist, sh_out):
        b = lax.axis_index("c")
        s = lax.axis_index("s")
        base = s * per_w
        lane = lax.iota(jnp.int32, 16)
        ones16 = jnp.ones((16,), jnp.int32)

        pltpu.sync_copy(codes_hbm.at[b, pl.ds(base, per_w)], keys_v)

        def zero_body(i, _):
            hist_v[pl.ds(i * 16, 16)] = jnp.zeros((16,), jnp.int32)
            return 0
        lax.fori_loop(0, 16 * _NBINS // 16, zero_body, 0)

        def hist_body(i, _):
            k16 = keys_v[pl.ds(i * 16, 16)]
            plsc.addupdate_scatter(hist_v, [lane * _NBINS + k16], ones16)
            return 0
        lax.fori_loop(0, nvec, hist_body, 0)

        for j in range(_NBINS // 16):
            acc = jnp.zeros((16,), jnp.int32)
            for l in range(16):
                acc = acc + hist_v[pl.ds(l * _NBINS + j * 16, 16)]
            histred_v[pl.ds(j * 16, 16)] = acc

        pltpu.sync_copy(histred_v, sh_hist.at[s])
        plsc.subcore_barrier()
        pltpu.sync_copy(sh_hist, allhist_v)

        carry = jnp.int32(0)
        for j in range(_NBINS // 16):
            tot = jnp.zeros((16,), jnp.int32)
            mine = jnp.zeros((16,), jnp.int32)
            for l in range(_NSUB):
                row = allhist_v[l, pl.ds(j * 16, 16)]
                tot = tot + row
                mine = mine + row * jnp.where(l < s, jnp.int32(1), jnp.int32(0))
            inc = plsc.cumsum(tot)
            offs_v[pl.ds(j * 16, 16)] = (inc - tot) + mine + carry
            carry = carry + jnp.sum(tot)

        def rank_body(i, _):
            k16 = keys_v[pl.ds(i * 16, 16)]
            dst16 = jnp.zeros((16,), jnp.int32)
            for l in range(16):
                m = lane == l
                d = plsc.load_gather(offs_v, [k16])
                dst16 = jnp.where(m, d, dst16)
                plsc.addupdate_scatter(offs_v, [k16], ones16, mask=m)
            dst_v[pl.ds(i * 16, 16)] = dst16
            pos_v[pl.ds(i * 16, 16)] = base + i * 16 + lane
            return 0
        lax.fori_loop(0, nvec, rank_body, 0)

        pltpu.sync_copy(pos_v, sh_out.at[dst_v])
        plsc.subcore_barrier()
        pltpu.sync_copy(sh_out.at[pl.ds(base, per_w)], out_hbm.at[b, pl.ds(base, per_w)])

    return k(codes)


def _attn_body(rows_ref, fc2_b, out_ref):
    K = rows_ref.shape[1] // _CHUNK
    eye = jnp.eye(_CHUNK, dtype=jnp.float32)
    zpad = jnp.zeros((_CHUNK, _OROW - _CHANNELS - 1), dtype=jnp.float32)

    def chunk(start):
        return rows_ref[0, pl.ds(start, _CHUNK), :]

    t0 = _CR + _CHANNELS

    def body(k, _):
        cur = chunk(k * _CHUNK)
        prv = chunk(lax.rem(k + K - 1, K) * _CHUNK)
        nxt = chunk(lax.rem(k + 1, K) * _CHUNK)
        xq = cur[:, :_CR]

        def nrm(t):
            x = t[:, :_CR]
            n = jnp.sqrt(jnp.sum(x * x, axis=-1, keepdims=True))
            return x / jnp.maximum(n, 5e-05)

        xm = jnp.concatenate([nrm(cur), nrm(prv), nrm(nxt)], axis=0)
        yc = jnp.concatenate([cur[:, _CR:t0],
                              prv[:, _CR:t0],
                              nxt[:, _CR:t0]], axis=0)
        tc = jnp.concatenate([cur[:, t0:t0 + _CHUNK],
                              prv[:, t0:t0 + _CHUNK],
                              nxt[:, t0:t0 + _CHUNK]], axis=0)

        raw_t = lax.dot_general(xm, xq, (((1,), (1,)), ((), ())),
                                preferred_element_type=jnp.float32) + tc + fc2_b[...]
        m = jnp.max(raw_t, axis=0, keepdims=True)
        e = jnp.exp(raw_t - m)
        s = jnp.sum(e, axis=0, keepdims=True)
        ret = lax.dot_general(e, yc, (((0,), (0,)), ((), ())),
                              preferred_element_type=jnp.float32)
        bsms = m + jnp.log(s)
        bscol = lax.dot_general(eye, bsms, (((1,), (1,)), ((), ())),
                                preferred_element_type=jnp.float32)
        scol = lax.dot_general(eye, s, (((1,), (1,)), ((), ())),
                               preferred_element_type=jnp.float32)
        ret = ret * (1.0 / scol)
        out_ref[0, pl.ds(k * _CHUNK, _CHUNK), :] = jnp.concatenate(
            [ret, bscol, zpad], axis=1)
        return 0

    lax.fori_loop(0, K, body, 0)


def _bucket_attention(rows_sorted, fc2_b, G, LH):
    return pl.pallas_call(
        _attn_body,
        grid=(G,),
        in_specs=[
            pl.BlockSpec((1, LH, _ROW), lambda h: (h, 0, 0)),
            pl.BlockSpec((1, _CHUNK), lambda h: (0, 0)),
        ],
        out_specs=pl.BlockSpec((1, LH, _OROW), lambda h: (h, 0, 0)),
        out_shape=jax.ShapeDtypeStruct((G, LH, _OROW), jnp.float32),
    )(rows_sorted, fc2_b.reshape(1, -1))


def _conv2d_relu(x, w, b):
    y = lax.conv_general_dilated(x, w, window_strides=(1, 1), padding='SAME',
                                 dimension_numbers=('NCHW', 'OIHW', 'NCHW'))
    return jax.nn.relu(y + b[None, :, None, None])


def kernel(input, cm_w, cm_b, ca_w, ca_b, cf_w, cf_b, fc1_w, fc1_b, fc2_w, fc2_b, random_rotations):
    x_nchw = jnp.transpose(input, (0, 3, 1, 2))
    N, _, H, W = x_nchw.shape
    L = H * W
    x_embed = _conv2d_relu(x_nchw, cm_w, cm_b).reshape(N, -1, L).transpose(0, 2, 1)
    y_embed = _conv2d_relu(x_nchw, ca_w, ca_b).reshape(N, -1, L).transpose(0, 2, 1)
    fc_embed = _conv2d_relu(x_nchw, cf_w, cf_b).reshape(N, -1, L).transpose(0, 2, 1)
    C = x_embed.shape[-1]
    hb = min(L // _CHUNK + (L // _CHUNK) % 2, 128)

    rot = random_rotations.reshape(C, _N_HASHES, hb)
    rotated = jnp.einsum('btf,fhi->bhti', x_embed, rot)
    hash_codes = jnp.argmax(rotated, axis=-1)
    offsets = (jnp.arange(_N_HASHES) * hb).reshape(1, -1, 1)
    hash_codes = (hash_codes + offsets).reshape(N, -1)
    indices = _sc_counting_sort(hash_codes.astype(jnp.int32), N, hash_codes.shape[1])

    table = _build_table(
        x_embed.reshape(N * L, C), y_embed.reshape(N * L, _CHANNELS),
        fc_embed.reshape(N * L, _CHANNELS), fc1_w, fc1_b, fc2_w)

    HL = _N_HASHES * L
    gidx = (indices % L + (jnp.arange(N) * L)[:, None]).astype(jnp.int32)
    rows_sorted = _sc_gather(table, gidx.reshape(_NW, -1, _SCCHUNK), _ROW)

    G = N * _N_HASHES
    LH = L
    out80 = _bucket_attention(rows_sorted.reshape(G, LH, _ROW), fc2_b, G, LH)

    gdst = (indices + (jnp.arange(N) * HL)[:, None]).astype(jnp.int32)
    unsorted = _sc_scatter(out80.reshape(N * HL, _OROW), gdst.reshape(_NW, -1, _SCCHUNK), _OROW)

    ret = unsorted[:, :_CHANNELS].reshape(N, _N_HASHES, L, _CHANNELS)
    bscore = unsorted[:, _CHANNELS].reshape(N, _N_HASHES, L, 1)
    probs = jax.nn.softmax(bscore, axis=1)
    ret = jnp.sum(ret * probs, axis=1).reshape(N, H, W, -1)
    return ret + input


# speedup vs baseline: 1.3398x; 1.1282x over previous
import functools
import jax
import jax.numpy as jnp
from jax import lax
from jax.experimental import pallas as pl
from jax.experimental.pallas import tpu as pltpu
from jax.experimental.pallas import tpu_sc as plsc

_N_HASHES = 4
_CHANNELS = 64
_REDUCTION = 4
_CHUNK = 144
_CR = _CHANNELS // _REDUCTION
_ROW = 256
_OROW = 128

_NW = 32
_SCCHUNK = 128


def _embed_body(x_ref, y_ref, f_ref, fc1_w, fc1_b, fc2_w, out_ref):
    BLK = x_ref.shape[0]
    h1 = jax.nn.relu(
        lax.dot_general(f_ref[...], fc1_w[...], (((1,), (1,)), ((), ())),
                        preferred_element_type=jnp.float32) + fc1_b[...])
    t = lax.dot_general(h1, fc2_w[...], (((1,), (1,)), ((), ())),
                        preferred_element_type=jnp.float32)
    pad = jnp.zeros((BLK, _ROW - _CR - _CHANNELS - _CHUNK), jnp.float32)
    out_ref[...] = jnp.concatenate([x_ref[...], y_ref[...], t, pad], axis=1)


def _build_table(x_embed, y_embed, f_embed, fc1_w, fc1_b, fc2_w):
    NL = x_embed.shape[0]
    BLK = 1024
    grid = (NL // BLK,)
    return pl.pallas_call(
        _embed_body,
        grid=grid,
        in_specs=[
            pl.BlockSpec((BLK, _CR), lambda i: (i, 0)),
            pl.BlockSpec((BLK, _CHANNELS), lambda i: (i, 0)),
            pl.BlockSpec((BLK, _CHANNELS), lambda i: (i, 0)),
            pl.BlockSpec((_CHUNK, _CHANNELS), lambda i: (0, 0)),
            pl.BlockSpec((1, _CHUNK), lambda i: (0, 0)),
            pl.BlockSpec((_CHUNK, _CHUNK), lambda i: (0, 0)),
        ],
        out_specs=pl.BlockSpec((BLK, _ROW), lambda i: (i, 0)),
        out_shape=jax.ShapeDtypeStruct((NL, _ROW), jnp.float32),
    )(x_embed, y_embed, f_embed, fc1_w, fc1_b.reshape(1, -1), fc2_w)


_RB = 8


def _front_body(xm1_ref, x_ref, xp1_ref, w_ref, b_ref, fc1_w, fc1_b, fc2_w,
                rot_ref, table_ref, codes_ref):
    i = pl.program_id(1)
    nrow = pl.num_programs(1)
    Wd = x_ref.shape[2]
    BLKP = _RB * Wd
    top = jnp.where(i == 0, 0.0, 1.0)
    bot = jnp.where(i == nrow - 1, 0.0, 1.0)
    strip = jnp.concatenate(
        [xm1_ref[0, _RB - 1:_RB] * top, x_ref[0], xp1_ref[0, 0:1] * bot], axis=0)
    zcol = jnp.zeros((_RB + 2, 1, strip.shape[2]), jnp.float32)
    stripx = jnp.concatenate([zcol, strip, zcol], axis=1)

    acc = jnp.broadcast_to(b_ref[...], (BLKP, _CHUNK))
    for dy in range(3):
        for dx in range(3):
            sl = stripx[dy:dy + _RB, dx:dx + Wd, :].reshape(BLKP, -1)
            acc = acc + lax.dot_general(sl, w_ref[dy * 3 + dx],
                                        (((1,), (0,)), ((), ())),
                                        preferred_element_type=jnp.float32)
    e = jax.nn.relu(acc)
    x_e = e[:, :_CR]
    y_e = e[:, _CR:_CR + _CHANNELS]
    f_e = e[:, _CR + _CHANNELS:]

    h1 = jax.nn.relu(
        lax.dot_general(f_e, fc1_w[...], (((1,), (1,)), ((), ())),
                        preferred_element_type=jnp.float32) + fc1_b[...])
    t = lax.dot_general(h1, fc2_w[...], (((1,), (1,)), ((), ())),
                        preferred_element_type=jnp.float32)
    pad = jnp.zeros((BLKP, _ROW - _CR - _CHANNELS - _CHUNK), jnp.float32)
    table_ref[0] = jnp.concatenate([x_e, y_e, t, pad], axis=1)

    r = lax.dot_general(x_e, rot_ref[...], (((1,), (0,)), ((), ())),
                        preferred_element_type=jnp.float32)
    iota = lax.broadcasted_iota(jnp.int32, (BLKP, 64), 1)
    cols = []
    for h in range(_N_HASHES):
        sub = r[:, h * 64:(h + 1) * 64]
        m = jnp.max(sub, axis=1, keepdims=True)
        idx = jnp.min(jnp.where(sub == m, iota, 64), axis=1, keepdims=True)
        cols.append(idx + h * 64)
    codes_ref[0] = jnp.concatenate(cols, axis=1)


def _front(x_img, cm_w, cm_b, ca_w, ca_b, cf_w, cf_b, fc1_w, fc1_b, fc2_w, rot):
    N, H, Wd, Cin = x_img.shape
    w_all = jnp.transpose(jnp.concatenate([cm_w, ca_w, cf_w], axis=0),
                          (2, 3, 1, 0)).reshape(9, Cin, _CHUNK)
    b_all = jnp.concatenate([cm_b, ca_b, cf_b]).reshape(1, _CHUNK)
    nrow = H // _RB
    grid = (N, nrow)

    def rb(off):
        return lambda n, i: (n, (i + off) % nrow, 0, 0)

    def full(shape):
        return pl.BlockSpec(shape, lambda n, i: tuple(0 for _ in shape))

    table, codes = pl.pallas_call(
        _front_body,
        grid=grid,
        in_specs=[
            pl.BlockSpec((1, _RB, Wd, Cin), rb(-1)),
            pl.BlockSpec((1, _RB, Wd, Cin), rb(0)),
            pl.BlockSpec((1, _RB, Wd, Cin), rb(1)),
            full((9, Cin, _CHUNK)),
            full((1, _CHUNK)),
            full((_CHUNK, _CHANNELS)),
            full((1, _CHUNK)),
            full((_CHUNK, _CHUNK)),
            full((_CR, _N_HASHES * 64)),
        ],
        out_specs=[
            pl.BlockSpec((1, _RB * Wd, _ROW), lambda n, i: (n, i, 0)),
            pl.BlockSpec((1, _RB * Wd, _N_HASHES), lambda n, i: (n, i, 0)),
        ],
        out_shape=[
            jax.ShapeDtypeStruct((N, H * Wd, _ROW), jnp.float32),
            jax.ShapeDtypeStruct((N, H * Wd, _N_HASHES), jnp.int32),
        ],
    )(x_img, x_img, x_img, w_all, b_all, fc1_w, fc1_b.reshape(1, -1), fc2_w, rot)
    return table.reshape(N * H * Wd, _ROW), codes


def _sc_gather(table, gidx3d, D):
    B = _NW * gidx3d.shape[1] * _SCCHUNK
    per_w = B // _NW
    n_ch = per_w // _SCCHUNK
    mesh = plsc.VectorSubcoreMesh(core_axis_name="c", subcore_axis_name="s")

    @functools.partial(
        pl.kernel, mesh=mesh,
        out_type=jax.ShapeDtypeStruct((B, D), jnp.float32),
        scratch_types=[
            pltpu.VMEM((n_ch, _SCCHUNK), jnp.int32),
            pltpu.VMEM((_SCCHUNK, D), jnp.float32),
            pltpu.SemaphoreType.DMA,
        ],
    )
    def k(table_hbm, idx_hbm, out_hbm, idx_v, rows_v, sem):
        wid = lax.axis_index("s") * 2 + lax.axis_index("c")
        pltpu.sync_copy(idx_hbm.at[wid], idx_v)

        def body(j, _):
            pltpu.async_copy(table_hbm.at[idx_v.at[j]], rows_v, sem).wait()
            base = wid * per_w + j * _SCCHUNK
            pltpu.sync_copy(rows_v, out_hbm.at[pl.ds(base, _SCCHUNK)])
            return 0

        lax.fori_loop(0, n_ch, body, 0)

    return k(table, gidx3d)


def _sc_scatter(rows, gdst3d, D):
    B = _NW * gdst3d.shape[1] * _SCCHUNK
    per_w = B // _NW
    n_ch = per_w // _SCCHUNK
    mesh = plsc.VectorSubcoreMesh(core_axis_name="c", subcore_axis_name="s")

    @functools.partial(
        pl.kernel, mesh=mesh,
        out_type=jax.ShapeDtypeStruct((B, D), jnp.float32),
        scratch_types=[
            pltpu.VMEM((n_ch, _SCCHUNK), jnp.int32),
            pltpu.VMEM((_SCCHUNK, D), jnp.float32),
            pltpu.SemaphoreType.DMA,
        ],
    )
    def k(rows_hbm, idx_hbm, out_hbm, idx_v, rows_v, sem):
        wid = lax.axis_index("s") * 2 + lax.axis_index("c")
        pltpu.sync_copy(idx_hbm.at[wid], idx_v)

        def body(j, _):
            base = wid * per_w + j * _SCCHUNK
            pltpu.sync_copy(rows_hbm.at[pl.ds(base, _SCCHUNK)], rows_v)
            pltpu.async_copy(rows_v, out_hbm.at[idx_v.at[j]], sem).wait()
            return 0

        lax.fori_loop(0, n_ch, body, 0)

    return k(rows, gdst3d)


_NBINS = 256
_NSUB = 16


def _sc_counting_sort(codes, NB, Ltot):
    per_w = Ltot // _NSUB
    nvec = per_w // 16
    mesh = plsc.VectorSubcoreMesh(core_axis_name="c", subcore_axis_name="s")

    @functools.partial(
        pl.kernel, mesh=mesh,
        out_type=jax.ShapeDtypeStruct((NB, Ltot), jnp.int32),
        compiler_params=pltpu.CompilerParams(needs_layout_passes=False),
        scratch_types=[
            pltpu.VMEM((per_w,), jnp.int32),
            pltpu.VMEM((16 * _NBINS,), jnp.int32),
            pltpu.VMEM((_NBINS,), jnp.int32),
            pltpu.VMEM((_NSUB, _NBINS), jnp.int32),
            pltpu.VMEM((_NBINS,), jnp.int32),
            pltpu.VMEM((per_w,), jnp.int32),
            pltpu.VMEM((per_w,), jnp.int32),
            pltpu.VMEM_SHARED((_NSUB, _NBINS), jnp.int32),
            pltpu.VMEM_SHARED((Ltot,), jnp.int32),
        ],
    )
    def k(codes_hbm, out_hbm, keys_v, hist_v, histred_v, allhist_v, offs_v,
          dst_v, pos_v, sh_hist, sh_out):
        b = lax.axis_index("c")
        s = lax.axis_index("s")
        base = s * per_w
        lane = lax.iota(jnp.int32, 16)
        ones16 = jnp.ones((16,), jnp.int32)

        pltpu.sync_copy(codes_hbm.at[b, pl.ds(base, per_w)], keys_v)

        def zero_body(i, _):
            hist_v[pl.ds(i * 16, 16)] = jnp.zeros((16,), jnp.int32)
            return 0
        lax.fori_loop(0, 16 * _NBINS // 16, zero_body, 0)

        def hist_body(i, _):
            k16 = keys_v[pl.ds(i * 16, 16)]
            plsc.addupdate_scatter(hist_v, [lane * _NBINS + k16], ones16)
            return 0
        lax.fori_loop(0, nvec, hist_body, 0)

        for j in range(_NBINS // 16):
            acc = jnp.zeros((16,), jnp.int32)
            for l in range(16):
                acc = acc + hist_v[pl.ds(l * _NBINS + j * 16, 16)]
            histred_v[pl.ds(j * 16, 16)] = acc

        pltpu.sync_copy(histred_v, sh_hist.at[s])
        plsc.subcore_barrier()
        pltpu.sync_copy(sh_hist, allhist_v)

        carry = jnp.int32(0)
        for j in range(_NBINS // 16):
            tot = jnp.zeros((16,), jnp.int32)
            mine = jnp.zeros((16,), jnp.int32)
            for l in range(_NSUB):
                row = allhist_v[l, pl.ds(j * 16, 16)]
                tot = tot + row
                mine = mine + row * jnp.where(l < s, jnp.int32(1), jnp.int32(0))
            inc = plsc.cumsum(tot)
            offs_v[pl.ds(j * 16, 16)] = (inc - tot) + mine + carry
            carry = carry + jnp.sum(tot)

        def rank_body(i, _):
            k16 = keys_v[pl.ds(i * 16, 16)]
            dst16 = jnp.zeros((16,), jnp.int32)
            for l in range(16):
                m = lane == l
                d = plsc.load_gather(offs_v, [k16])
                dst16 = jnp.where(m, d, dst16)
                plsc.addupdate_scatter(offs_v, [k16], ones16, mask=m)
            dst_v[pl.ds(i * 16, 16)] = dst16
            pos_v[pl.ds(i * 16, 16)] = base + i * 16 + lane
            return 0
        lax.fori_loop(0, nvec, rank_body, 0)

        pltpu.sync_copy(pos_v, sh_out.at[dst_v])
        plsc.subcore_barrier()
        pltpu.sync_copy(sh_out.at[pl.ds(base, per_w)], out_hbm.at[b, pl.ds(base, per_w)])

    return k(codes)


def _attn_body(rows_ref, fc2_b, out_ref):
    K = rows_ref.shape[1] // _CHUNK
    eye = jnp.eye(_CHUNK, dtype=jnp.float32)
    zpad = jnp.zeros((_CHUNK, _OROW - _CHANNELS - 1), dtype=jnp.float32)

    def chunk(start):
        return rows_ref[0, pl.ds(start, _CHUNK), :]

    t0 = _CR + _CHANNELS

    def body(k, _):
        cur = chunk(k * _CHUNK)
        prv = chunk(lax.rem(k + K - 1, K) * _CHUNK)
        nxt = chunk(lax.rem(k + 1, K) * _CHUNK)
        xq = cur[:, :_CR]

        def nrm(t):
            x = t[:, :_CR]
            n = jnp.sqrt(jnp.sum(x * x, axis=-1, keepdims=True))
            return x / jnp.maximum(n, 5e-05)

        xm = jnp.concatenate([nrm(cur), nrm(prv), nrm(nxt)], axis=0)
        yc = jnp.concatenate([cur[:, _CR:t0],
                              prv[:, _CR:t0],
                              nxt[:, _CR:t0]], axis=0)
        tc = jnp.concatenate([cur[:, t0:t0 + _CHUNK],
                              prv[:, t0:t0 + _CHUNK],
                              nxt[:, t0:t0 + _CHUNK]], axis=0)

        raw_t = lax.dot_general(xm, xq, (((1,), (1,)), ((), ())),
                                preferred_element_type=jnp.float32) + tc + fc2_b[...]
        m = jnp.max(raw_t, axis=0, keepdims=True)
        e = jnp.exp(raw_t - m)
        s = jnp.sum(e, axis=0, keepdims=True)
        ret = lax.dot_general(e, yc, (((0,), (0,)), ((), ())),
                              preferred_element_type=jnp.float32)
        bsms = m + jnp.log(s)
        bscol = lax.dot_general(eye, bsms, (((1,), (1,)), ((), ())),
                                preferred_element_type=jnp.float32)
        scol = lax.dot_general(eye, s, (((1,), (1,)), ((), ())),
                               preferred_element_type=jnp.float32)
        ret = ret * (1.0 / scol)
        out_ref[0, pl.ds(k * _CHUNK, _CHUNK), :] = jnp.concatenate(
            [ret, bscol, zpad], axis=1)
        return 0

    lax.fori_loop(0, K, body, 0)


def _bucket_attention(rows_sorted, fc2_b, G, LH):
    return pl.pallas_call(
        _attn_body,
        grid=(G,),
        in_specs=[
            pl.BlockSpec((1, LH, _ROW), lambda h: (h, 0, 0)),
            pl.BlockSpec((1, _CHUNK), lambda h: (0, 0)),
        ],
        out_specs=pl.BlockSpec((1, LH, _OROW), lambda h: (h, 0, 0)),
        out_shape=jax.ShapeDtypeStruct((G, LH, _OROW), jnp.float32),
    )(rows_sorted, fc2_b.reshape(1, -1))


def kernel(input, cm_w, cm_b, ca_w, ca_b, cf_w, cf_b, fc1_w, fc1_b, fc2_w, fc2_b, random_rotations):
    N, H, W, _ = input.shape
    L = H * W

    table, codes = _front(input, cm_w, cm_b, ca_w, ca_b, cf_w, cf_b,
                          fc1_w, fc1_b, fc2_w, random_rotations)

    hash_codes = codes.transpose(0, 2, 1).reshape(N, -1)
    indices = _sc_counting_sort(hash_codes.astype(jnp.int32), N, hash_codes.shape[1])

    HL = _N_HASHES * L
    gidx = (indices % L + (jnp.arange(N) * L)[:, None]).astype(jnp.int32)
    rows_sorted = _sc_gather(table, gidx.reshape(_NW, -1, _SCCHUNK), _ROW)

    G = N * _N_HASHES
    LH = L
    out80 = _bucket_attention(rows_sorted.reshape(G, LH, _ROW), fc2_b, G, LH)

    gdst = (indices + (jnp.arange(N) * HL)[:, None]).astype(jnp.int32)
    unsorted = _sc_scatter(out80.reshape(N * HL, _OROW), gdst.reshape(_NW, -1, _SCCHUNK), _OROW)

    ret = unsorted[:, :_CHANNELS].reshape(N, _N_HASHES, L, _CHANNELS)
    bscore = unsorted[:, _CHANNELS].reshape(N, _N_HASHES, L, 1)
    probs = jax.nn.softmax(bscore, axis=1)
    ret = jnp.sum(ret * probs, axis=1).reshape(N, H, W, -1)
    return ret + input


# speedup vs baseline: 1.3526x; 1.0095x over previous
import functools
import jax
import jax.numpy as jnp
from jax import lax
from jax.experimental import pallas as pl
from jax.experimental.pallas import tpu as pltpu
from jax.experimental.pallas import tpu_sc as plsc

_N_HASHES = 4
_CHANNELS = 64
_REDUCTION = 4
_CHUNK = 144
_CR = _CHANNELS // _REDUCTION
_ROW = 256
_OROW = 128

_NW = 32
_SCCHUNK = 128


def _embed_body(x_ref, y_ref, f_ref, fc1_w, fc1_b, fc2_w, out_ref):
    BLK = x_ref.shape[0]
    h1 = jax.nn.relu(
        lax.dot_general(f_ref[...], fc1_w[...], (((1,), (1,)), ((), ())),
                        preferred_element_type=jnp.float32) + fc1_b[...])
    t = lax.dot_general(h1, fc2_w[...], (((1,), (1,)), ((), ())),
                        preferred_element_type=jnp.float32)
    pad = jnp.zeros((BLK, _ROW - _CR - _CHANNELS - _CHUNK), jnp.float32)
    out_ref[...] = jnp.concatenate([x_ref[...], y_ref[...], t, pad], axis=1)


def _build_table(x_embed, y_embed, f_embed, fc1_w, fc1_b, fc2_w):
    NL = x_embed.shape[0]
    BLK = 1024
    grid = (NL // BLK,)
    return pl.pallas_call(
        _embed_body,
        grid=grid,
        in_specs=[
            pl.BlockSpec((BLK, _CR), lambda i: (i, 0)),
            pl.BlockSpec((BLK, _CHANNELS), lambda i: (i, 0)),
            pl.BlockSpec((BLK, _CHANNELS), lambda i: (i, 0)),
            pl.BlockSpec((_CHUNK, _CHANNELS), lambda i: (0, 0)),
            pl.BlockSpec((1, _CHUNK), lambda i: (0, 0)),
            pl.BlockSpec((_CHUNK, _CHUNK), lambda i: (0, 0)),
        ],
        out_specs=pl.BlockSpec((BLK, _ROW), lambda i: (i, 0)),
        out_shape=jax.ShapeDtypeStruct((NL, _ROW), jnp.float32),
    )(x_embed, y_embed, f_embed, fc1_w, fc1_b.reshape(1, -1), fc2_w)


_RB = 16


def _front_body(xm1_ref, x_ref, xp1_ref, w_ref, b_ref, fc1_w, fc1_b, fc2_w,
                rot_ref, table_ref, codes_ref):
    i = pl.program_id(1)
    nrow = pl.num_programs(1)
    Wd = x_ref.shape[2]
    BLKP = _RB * Wd
    top = jnp.where(i == 0, 0.0, 1.0)
    bot = jnp.where(i == nrow - 1, 0.0, 1.0)
    strip = jnp.concatenate(
        [xm1_ref[0, _RB - 1:_RB] * top, x_ref[0], xp1_ref[0, 0:1] * bot], axis=0)
    zcol = jnp.zeros((_RB + 2, 1, strip.shape[2]), jnp.float32)
    stripx = jnp.concatenate([zcol, strip, zcol], axis=1)

    acc = jnp.broadcast_to(b_ref[...], (BLKP, _CHUNK))
    for dy in range(3):
        for dx in range(3):
            sl = stripx[dy:dy + _RB, dx:dx + Wd, :].reshape(BLKP, -1)
            acc = acc + lax.dot_general(sl, w_ref[dy * 3 + dx],
                                        (((1,), (0,)), ((), ())),
                                        preferred_element_type=jnp.float32)
    e = jax.nn.relu(acc)
    x_e = e[:, :_CR]
    y_e = e[:, _CR:_CR + _CHANNELS]
    f_e = e[:, _CR + _CHANNELS:]

    h1 = jax.nn.relu(
        lax.dot_general(f_e, fc1_w[...], (((1,), (1,)), ((), ())),
                        preferred_element_type=jnp.float32) + fc1_b[...])
    t = lax.dot_general(h1, fc2_w[...], (((1,), (1,)), ((), ())),
                        preferred_element_type=jnp.float32)
    pad = jnp.zeros((BLKP, _ROW - _CR - _CHANNELS - _CHUNK), jnp.float32)
    table_ref[0] = jnp.concatenate([x_e, y_e, t, pad], axis=1)

    r = lax.dot_general(x_e, rot_ref[...], (((1,), (0,)), ((), ())),
                        preferred_element_type=jnp.float32)
    iota = lax.broadcasted_iota(jnp.int32, (BLKP, 64), 1)
    cols = []
    for h in range(_N_HASHES):
        sub = r[:, h * 64:(h + 1) * 64]
        m = jnp.max(sub, axis=1, keepdims=True)
        idx = jnp.min(jnp.where(sub == m, iota, 64), axis=1, keepdims=True)
        cols.append(idx + h * 64)
    codes_ref[0] = jnp.concatenate(cols, axis=1)


def _front(x_img, cm_w, cm_b, ca_w, ca_b, cf_w, cf_b, fc1_w, fc1_b, fc2_w, rot):
    N, H, Wd, Cin = x_img.shape
    w_all = jnp.transpose(jnp.concatenate([cm_w, ca_w, cf_w], axis=0),
                          (2, 3, 1, 0)).reshape(9, Cin, _CHUNK)
    b_all = jnp.concatenate([cm_b, ca_b, cf_b]).reshape(1, _CHUNK)
    nrow = H // _RB
    grid = (N, nrow)

    def rb(off):
        return lambda n, i: (n, (i + off) % nrow, 0, 0)

    def full(shape):
        return pl.BlockSpec(shape, lambda n, i: tuple(0 for _ in shape))

    table, codes = pl.pallas_call(
        _front_body,
        grid=grid,
        in_specs=[
            pl.BlockSpec((1, _RB, Wd, Cin), rb(-1)),
            pl.BlockSpec((1, _RB, Wd, Cin), rb(0)),
            pl.BlockSpec((1, _RB, Wd, Cin), rb(1)),
            full((9, Cin, _CHUNK)),
            full((1, _CHUNK)),
            full((_CHUNK, _CHANNELS)),
            full((1, _CHUNK)),
            full((_CHUNK, _CHUNK)),
            full((_CR, _N_HASHES * 64)),
        ],
        out_specs=[
            pl.BlockSpec((1, _RB * Wd, _ROW), lambda n, i: (n, i, 0)),
            pl.BlockSpec((1, _RB * Wd, _N_HASHES), lambda n, i: (n, i, 0)),
        ],
        out_shape=[
            jax.ShapeDtypeStruct((N, H * Wd, _ROW), jnp.float32),
            jax.ShapeDtypeStruct((N, H * Wd, _N_HASHES), jnp.int32),
        ],
    )(x_img, x_img, x_img, w_all, b_all, fc1_w, fc1_b.reshape(1, -1), fc2_w, rot)
    return table.reshape(N * H * Wd, _ROW), codes


def _sc_gather(table, gidx3d, D):
    B = _NW * gidx3d.shape[1] * _SCCHUNK
    per_w = B // _NW
    n_ch = per_w // _SCCHUNK
    mesh = plsc.VectorSubcoreMesh(core_axis_name="c", subcore_axis_name="s")

    @functools.partial(
        pl.kernel, mesh=mesh,
        out_type=jax.ShapeDtypeStruct((B, D), jnp.float32),
        scratch_types=[
            pltpu.VMEM((n_ch, _SCCHUNK), jnp.int32),
            pltpu.VMEM((_SCCHUNK, D), jnp.float32),
            pltpu.SemaphoreType.DMA,
        ],
    )
    def k(table_hbm, idx_hbm, out_hbm, idx_v, rows_v, sem):
        wid = lax.axis_index("s") * 2 + lax.axis_index("c")
        pltpu.sync_copy(idx_hbm.at[wid], idx_v)

        def body(j, _):
            pltpu.async_copy(table_hbm.at[idx_v.at[j]], rows_v, sem).wait()
            base = wid * per_w + j * _SCCHUNK
            pltpu.sync_copy(rows_v, out_hbm.at[pl.ds(base, _SCCHUNK)])
            return 0

        lax.fori_loop(0, n_ch, body, 0)

    return k(table, gidx3d)


def _sc_scatter(rows, gdst3d, D):
    B = _NW * gdst3d.shape[1] * _SCCHUNK
    per_w = B // _NW
    n_ch = per_w // _SCCHUNK
    mesh = plsc.VectorSubcoreMesh(core_axis_name="c", subcore_axis_name="s")

    @functools.partial(
        pl.kernel, mesh=mesh,
        out_type=jax.ShapeDtypeStruct((B, D), jnp.float32),
        scratch_types=[
            pltpu.VMEM((n_ch, _SCCHUNK), jnp.int32),
            pltpu.VMEM((_SCCHUNK, D), jnp.float32),
            pltpu.SemaphoreType.DMA,
        ],
    )
    def k(rows_hbm, idx_hbm, out_hbm, idx_v, rows_v, sem):
        wid = lax.axis_index("s") * 2 + lax.axis_index("c")
        pltpu.sync_copy(idx_hbm.at[wid], idx_v)

        def body(j, _):
            base = wid * per_w + j * _SCCHUNK
            pltpu.sync_copy(rows_hbm.at[pl.ds(base, _SCCHUNK)], rows_v)
            pltpu.async_copy(rows_v, out_hbm.at[idx_v.at[j]], sem).wait()
            return 0

        lax.fori_loop(0, n_ch, body, 0)

    return k(rows, gdst3d)


_NBINS = 256
_NSUB = 16


def _sc_counting_sort(codes, NB, Ltot):
    per_w = Ltot // _NSUB
    nvec = per_w // 16
    mesh = plsc.VectorSubcoreMesh(core_axis_name="c", subcore_axis_name="s")

    @functools.partial(
        pl.kernel, mesh=mesh,
        out_type=jax.ShapeDtypeStruct((NB, Ltot), jnp.int32),
        compiler_params=pltpu.CompilerParams(needs_layout_passes=False),
        scratch_types=[
            pltpu.VMEM((per_w,), jnp.int32),
            pltpu.VMEM((16 * _NBINS,), jnp.int32),
            pltpu.VMEM((_NBINS,), jnp.int32),
            pltpu.VMEM((_NSUB, _NBINS), jnp.int32),
            pltpu.VMEM((_NBINS,), jnp.int32),
            pltpu.VMEM((per_w,), jnp.int32),
            pltpu.VMEM((per_w,), jnp.int32),
            pltpu.VMEM_SHARED((_NSUB, _NBINS), jnp.int32),
            pltpu.VMEM_SHARED((Ltot,), jnp.int32),
        ],
    )
    def k(codes_hbm, out_hbm, keys_v, hist_v, histred_v, allhist_v, offs_v,
          dst_v, pos_v, sh_hist, sh_out):
        b = lax.axis_index("c")
        s = lax.axis_index("s")
        base = s * per_w
        lane = lax.iota(jnp.int32, 16)
        ones16 = jnp.ones((16,), jnp.int32)

        pltpu.sync_copy(codes_hbm.at[b, pl.ds(base, per_w)], keys_v)

        def zero_body(i, _):
            hist_v[pl.ds(i * 16, 16)] = jnp.zeros((16,), jnp.int32)
            return 0
        lax.fori_loop(0, 16 * _NBINS // 16, zero_body, 0)

        def hist_body(i, _):
            k16 = keys_v[pl.ds(i * 16, 16)]
            plsc.addupdate_scatter(hist_v, [lane * _NBINS + k16], ones16)
            return 0
        lax.fori_loop(0, nvec, hist_body, 0)

        for j in range(_NBINS // 16):
            acc = jnp.zeros((16,), jnp.int32)
            for l in range(16):
                acc = acc + hist_v[pl.ds(l * _NBINS + j * 16, 16)]
            histred_v[pl.ds(j * 16, 16)] = acc

        pltpu.sync_copy(histred_v, sh_hist.at[s])
        plsc.subcore_barrier()
        pltpu.sync_copy(sh_hist, allhist_v)

        carry = jnp.int32(0)
        for j in range(_NBINS // 16):
            tot = jnp.zeros((16,), jnp.int32)
            mine = jnp.zeros((16,), jnp.int32)
            for l in range(_NSUB):
                row = allhist_v[l, pl.ds(j * 16, 16)]
                tot = tot + row
                mine = mine + row * jnp.where(l < s, jnp.int32(1), jnp.int32(0))
            inc = plsc.cumsum(tot)
            offs_v[pl.ds(j * 16, 16)] = (inc - tot) + mine + carry
            carry = carry + jnp.sum(tot)

        def rank_body(i, _):
            k16 = keys_v[pl.ds(i * 16, 16)]
            dst16 = jnp.zeros((16,), jnp.int32)
            for l in range(16):
                m = lane == l
                d = plsc.load_gather(offs_v, [k16])
                dst16 = jnp.where(m, d, dst16)
                plsc.addupdate_scatter(offs_v, [k16], ones16, mask=m)
            dst_v[pl.ds(i * 16, 16)] = dst16
            pos_v[pl.ds(i * 16, 16)] = base + i * 16 + lane
            return 0
        lax.fori_loop(0, nvec, rank_body, 0)

        pltpu.sync_copy(pos_v, sh_out.at[dst_v])
        plsc.subcore_barrier()
        pltpu.sync_copy(sh_out.at[pl.ds(base, per_w)], out_hbm.at[b, pl.ds(base, per_w)])

    return k(codes)


def _attn_body(rows_ref, fc2_b, out_ref):
    K = rows_ref.shape[1] // _CHUNK
    eye = jnp.eye(_CHUNK, dtype=jnp.float32)
    zpad = jnp.zeros((_CHUNK, _OROW - _CHANNELS - 1), dtype=jnp.float32)

    def chunk(start):
        return rows_ref[0, pl.ds(start, _CHUNK), :]

    t0 = _CR + _CHANNELS

    def body(k, _):
        cur = chunk(k * _CHUNK)
        prv = chunk(lax.rem(k + K - 1, K) * _CHUNK)
        nxt = chunk(lax.rem(k + 1, K) * _CHUNK)
        xq = cur[:, :_CR]

        def nrm(t):
            x = t[:, :_CR]
            n = jnp.sqrt(jnp.sum(x * x, axis=-1, keepdims=True))
            return x / jnp.maximum(n, 5e-05)

        xm = jnp.concatenate([nrm(cur), nrm(prv), nrm(nxt)], axis=0)
        yc = jnp.concatenate([cur[:, _CR:t0],
                              prv[:, _CR:t0],
                              nxt[:, _CR:t0]], axis=0)
        tc = jnp.concatenate([cur[:, t0:t0 + _CHUNK],
                              prv[:, t0:t0 + _CHUNK],
                              nxt[:, t0:t0 + _CHUNK]], axis=0)

        raw_t = lax.dot_general(xm, xq, (((1,), (1,)), ((), ())),
                                preferred_element_type=jnp.float32) + tc + fc2_b[...]
        m = jnp.max(raw_t, axis=0, keepdims=True)
        e = jnp.exp(raw_t - m)
        s = jnp.sum(e, axis=0, keepdims=True)
        ret = lax.dot_general(e, yc, (((0,), (0,)), ((), ())),
                              preferred_element_type=jnp.float32)
        bsms = m + jnp.log(s)
        bscol = lax.dot_general(eye, bsms, (((1,), (1,)), ((), ())),
                                preferred_element_type=jnp.float32)
        scol = lax.dot_general(eye, s, (((1,), (1,)), ((), ())),
                               preferred_element_type=jnp.float32)
        ret = ret * (1.0 / scol)
        out_ref[0, pl.ds(k * _CHUNK, _CHUNK), :] = jnp.concatenate(
            [ret, bscol, zpad], axis=1)
        return 0

    lax.fori_loop(0, K, body, 0)


def _bucket_attention(rows_sorted, fc2_b, G, LH):
    return pl.pallas_call(
        _attn_body,
        grid=(G,),
        in_specs=[
            pl.BlockSpec((1, LH, _ROW), lambda h: (h, 0, 0)),
            pl.BlockSpec((1, _CHUNK), lambda h: (0, 0)),
        ],
        out_specs=pl.BlockSpec((1, LH, _OROW), lambda h: (h, 0, 0)),
        out_shape=jax.ShapeDtypeStruct((G, LH, _OROW), jnp.float32),
    )(rows_sorted, fc2_b.reshape(1, -1))


def kernel(input, cm_w, cm_b, ca_w, ca_b, cf_w, cf_b, fc1_w, fc1_b, fc2_w, fc2_b, random_rotations):
    N, H, W, _ = input.shape
    L = H * W

    table, codes = _front(input, cm_w, cm_b, ca_w, ca_b, cf_w, cf_b,
                          fc1_w, fc1_b, fc2_w, random_rotations)

    hash_codes = codes.transpose(0, 2, 1).reshape(N, -1)
    indices = _sc_counting_sort(hash_codes.astype(jnp.int32), N, hash_codes.shape[1])

    HL = _N_HASHES * L
    gidx = (indices % L + (jnp.arange(N) * L)[:, None]).astype(jnp.int32)
    rows_sorted = _sc_gather(table, gidx.reshape(_NW, -1, _SCCHUNK), _ROW)

    G = N * _N_HASHES
    LH = L
    out80 = _bucket_attention(rows_sorted.reshape(G, LH, _ROW), fc2_b, G, LH)

    gdst = (indices + (jnp.arange(N) * HL)[:, None]).astype(jnp.int32)
    unsorted = _sc_scatter(out80.reshape(N * HL, _OROW), gdst.reshape(_NW, -1, _SCCHUNK), _OROW)

    ret = unsorted[:, :_CHANNELS].reshape(N, _N_HASHES, L, _CHANNELS)
    bscore = unsorted[:, _CHANNELS].reshape(N, _N_HASHES, L, 1)
    probs = jax.nn.softmax(bscore, axis=1)
    ret = jnp.sum(ret * probs, axis=1).reshape(N, H, W, -1)
    return ret + input


# speedup vs baseline: 1.3537x; 1.0008x over previous
import functools
import jax
import jax.numpy as jnp
from jax import lax
from jax.experimental import pallas as pl
from jax.experimental.pallas import tpu as pltpu
from jax.experimental.pallas import tpu_sc as plsc

_N_HASHES = 4
_CHANNELS = 64
_REDUCTION = 4
_CHUNK = 144
_CR = _CHANNELS // _REDUCTION
_ROW = 256
_OROW = 128

_NW = 32
_SCCHUNK = 128


_RB = 16


def _front_body(xm1_ref, x_ref, xp1_ref, w_ref, b_ref, fc1_w, fc1_b, fc2_w,
                rot_ref, table_ref, codes_ref):
    i = pl.program_id(1)
    nrow = pl.num_programs(1)
    Wd = x_ref.shape[2]
    BLKP = _RB * Wd
    top = jnp.where(i == 0, 0.0, 1.0)
    bot = jnp.where(i == nrow - 1, 0.0, 1.0)
    strip = jnp.concatenate(
        [xm1_ref[0, _RB - 1:_RB] * top, x_ref[0], xp1_ref[0, 0:1] * bot], axis=0)
    zcol = jnp.zeros((_RB + 2, 1, strip.shape[2]), jnp.float32)
    stripx = jnp.concatenate([zcol, strip, zcol], axis=1)

    acc = jnp.broadcast_to(b_ref[...], (BLKP, _CHUNK))
    for dy in range(3):
        for dx in range(3):
            sl = stripx[dy:dy + _RB, dx:dx + Wd, :].reshape(BLKP, -1)
            acc = acc + lax.dot_general(sl, w_ref[dy * 3 + dx],
                                        (((1,), (0,)), ((), ())),
                                        preferred_element_type=jnp.float32)
    e = jax.nn.relu(acc)
    x_e = e[:, :_CR]
    y_e = e[:, _CR:_CR + _CHANNELS]
    f_e = e[:, _CR + _CHANNELS:]

    h1 = jax.nn.relu(
        lax.dot_general(f_e, fc1_w[...], (((1,), (1,)), ((), ())),
                        preferred_element_type=jnp.float32) + fc1_b[...])
    t = lax.dot_general(h1, fc2_w[...], (((1,), (1,)), ((), ())),
                        preferred_element_type=jnp.float32)
    pad = jnp.zeros((BLKP, _ROW - _CR - _CHANNELS - _CHUNK), jnp.float32)
    table_ref[0] = jnp.concatenate([x_e, y_e, t, pad], axis=1)

    r = lax.dot_general(x_e, rot_ref[...], (((1,), (0,)), ((), ())),
                        preferred_element_type=jnp.float32)
    iota = lax.broadcasted_iota(jnp.int32, (BLKP, 64), 1)
    cols = []
    for h in range(_N_HASHES):
        sub = r[:, h * 64:(h + 1) * 64]
        m = jnp.max(sub, axis=1, keepdims=True)
        idx = jnp.min(jnp.where(sub == m, iota, 64), axis=1, keepdims=True)
        cols.append(idx + h * 64)
    codes_ref[0] = jnp.concatenate(cols, axis=1)


def _front(x_img, cm_w, cm_b, ca_w, ca_b, cf_w, cf_b, fc1_w, fc1_b, fc2_w, rot):
    N, H, Wd, Cin = x_img.shape
    w_all = jnp.transpose(jnp.concatenate([cm_w, ca_w, cf_w], axis=0),
                          (2, 3, 1, 0)).reshape(9, Cin, _CHUNK)
    b_all = jnp.concatenate([cm_b, ca_b, cf_b]).reshape(1, _CHUNK)
    nrow = H // _RB
    grid = (N, nrow)

    def rb(off):
        return lambda n, i: (n, (i + off) % nrow, 0, 0)

    def full(shape):
        return pl.BlockSpec(shape, lambda n, i: tuple(0 for _ in shape))

    table, codes = pl.pallas_call(
        _front_body,
        grid=grid,
        in_specs=[
            pl.BlockSpec((1, _RB, Wd, Cin), rb(-1)),
            pl.BlockSpec((1, _RB, Wd, Cin), rb(0)),
            pl.BlockSpec((1, _RB, Wd, Cin), rb(1)),
            full((9, Cin, _CHUNK)),
            full((1, _CHUNK)),
            full((_CHUNK, _CHANNELS)),
            full((1, _CHUNK)),
            full((_CHUNK, _CHUNK)),
            full((_CR, _N_HASHES * 64)),
        ],
        out_specs=[
            pl.BlockSpec((1, _RB * Wd, _ROW), lambda n, i: (n, i, 0)),
            pl.BlockSpec((1, _RB * Wd, _N_HASHES), lambda n, i: (n, i, 0)),
        ],
        out_shape=[
            jax.ShapeDtypeStruct((N, H * Wd, _ROW), jnp.float32),
            jax.ShapeDtypeStruct((N, H * Wd, _N_HASHES), jnp.int32),
        ],
    )(x_img, x_img, x_img, w_all, b_all, fc1_w, fc1_b.reshape(1, -1), fc2_w, rot)
    return table.reshape(N * H * Wd, _ROW), codes


def _sc_gather(table, gidx3d, D):
    B = _NW * gidx3d.shape[1] * _SCCHUNK
    per_w = B // _NW
    n_ch = per_w // _SCCHUNK
    mesh = plsc.VectorSubcoreMesh(core_axis_name="c", subcore_axis_name="s")

    @functools.partial(
        pl.kernel, mesh=mesh,
        out_type=jax.ShapeDtypeStruct((B, D), jnp.float32),
        scratch_types=[
            pltpu.VMEM((n_ch, _SCCHUNK), jnp.int32),
            pltpu.VMEM((_SCCHUNK, D), jnp.float32),
            pltpu.SemaphoreType.DMA,
        ],
    )
    def k(table_hbm, idx_hbm, out_hbm, idx_v, rows_v, sem):
        wid = lax.axis_index("s") * 2 + lax.axis_index("c")
        pltpu.sync_copy(idx_hbm.at[wid], idx_v)

        def body(j, _):
            pltpu.async_copy(table_hbm.at[idx_v.at[j]], rows_v, sem).wait()
            base = wid * per_w + j * _SCCHUNK
            pltpu.sync_copy(rows_v, out_hbm.at[pl.ds(base, _SCCHUNK)])
            return 0

        lax.fori_loop(0, n_ch, body, 0)

    return k(table, gidx3d)


def _sc_scatter(rows, gdst3d, D):
    B = _NW * gdst3d.shape[1] * _SCCHUNK
    per_w = B // _NW
    n_ch = per_w // _SCCHUNK
    mesh = plsc.VectorSubcoreMesh(core_axis_name="c", subcore_axis_name="s")

    @functools.partial(
        pl.kernel, mesh=mesh,
        out_type=jax.ShapeDtypeStruct((B, D), jnp.float32),
        scratch_types=[
            pltpu.VMEM((n_ch, _SCCHUNK), jnp.int32),
            pltpu.VMEM((_SCCHUNK, D), jnp.float32),
            pltpu.SemaphoreType.DMA,
        ],
    )
    def k(rows_hbm, idx_hbm, out_hbm, idx_v, rows_v, sem):
        wid = lax.axis_index("s") * 2 + lax.axis_index("c")
        pltpu.sync_copy(idx_hbm.at[wid], idx_v)

        def body(j, _):
            base = wid * per_w + j * _SCCHUNK
            pltpu.sync_copy(rows_hbm.at[pl.ds(base, _SCCHUNK)], rows_v)
            pltpu.async_copy(rows_v, out_hbm.at[idx_v.at[j]], sem).wait()
            return 0

        lax.fori_loop(0, n_ch, body, 0)

    return k(rows, gdst3d)


_NBINS = 256
_NSUB = 16


def _sc_counting_sort(codes, NB, Ltot):
    per_w = Ltot // _NSUB
    nvec = per_w // 16
    mesh = plsc.VectorSubcoreMesh(core_axis_name="c", subcore_axis_name="s")

    @functools.partial(
        pl.kernel, mesh=mesh,
        out_type=jax.ShapeDtypeStruct((NB, Ltot), jnp.int32),
        compiler_params=pltpu.CompilerParams(needs_layout_passes=False),
        scratch_types=[
            pltpu.VMEM((per_w,), jnp.int32),
            pltpu.VMEM((16 * _NBINS,), jnp.int32),
            pltpu.VMEM((_NBINS,), jnp.int32),
            pltpu.VMEM((_NSUB, _NBINS), jnp.int32),
            pltpu.VMEM((_NBINS,), jnp.int32),
            pltpu.VMEM((per_w,), jnp.int32),
            pltpu.VMEM((per_w,), jnp.int32),
            pltpu.VMEM_SHARED((_NSUB, _NBINS), jnp.int32),
            pltpu.VMEM_SHARED((Ltot,), jnp.int32),
        ],
    )
    def k(codes_hbm, out_hbm, keys_v, hist_v, histred_v, allhist_v, offs_v,
          dst_v, pos_v, sh_hist, sh_out):
        b = lax.axis_index("c")
        s = lax.axis_index("s")
        base = s * per_w
        lane = lax.iota(jnp.int32, 16)
        ones16 = jnp.ones((16,), jnp.int32)

        pltpu.sync_copy(codes_hbm.at[b, pl.ds(base, per_w)], keys_v)

        def zero_body(i, _):
            hist_v[pl.ds(i * 16, 16)] = jnp.zeros((16,), jnp.int32)
            return 0
        lax.fori_loop(0, 16 * _NBINS // 16, zero_body, 0)

        def hist_body(i, _):
            k16 = keys_v[pl.ds(i * 16, 16)]
            plsc.addupdate_scatter(hist_v, [lane * _NBINS + k16], ones16)
            return 0
        lax.fori_loop(0, nvec, hist_body, 0)

        for j in range(_NBINS // 16):
            acc = jnp.zeros((16,), jnp.int32)
            for l in range(16):
                acc = acc + hist_v[pl.ds(l * _NBINS + j * 16, 16)]
            histred_v[pl.ds(j * 16, 16)] = acc

        pltpu.sync_copy(histred_v, sh_hist.at[s])
        plsc.subcore_barrier()
        pltpu.sync_copy(sh_hist, allhist_v)

        carry = jnp.int32(0)
        for j in range(_NBINS // 16):
            tot = jnp.zeros((16,), jnp.int32)
            mine = jnp.zeros((16,), jnp.int32)
            for l in range(_NSUB):
                row = allhist_v[l, pl.ds(j * 16, 16)]
                tot = tot + row
                mine = mine + row * jnp.where(l < s, jnp.int32(1), jnp.int32(0))
            inc = plsc.cumsum(tot)
            offs_v[pl.ds(j * 16, 16)] = (inc - tot) + mine + carry
            carry = carry + jnp.sum(tot)

        def rank_body(i, _):
            k16 = keys_v[pl.ds(i * 16, 16)]
            dst16 = jnp.zeros((16,), jnp.int32)
            for l in range(16):
                m = lane == l
                d = plsc.load_gather(offs_v, [k16])
                dst16 = jnp.where(m, d, dst16)
                plsc.addupdate_scatter(offs_v, [k16], ones16, mask=m)
            dst_v[pl.ds(i * 16, 16)] = dst16
            pos_v[pl.ds(i * 16, 16)] = base + i * 16 + lane
            return 0
        lax.fori_loop(0, nvec, rank_body, 0)

        pltpu.sync_copy(pos_v, sh_out.at[dst_v])
        plsc.subcore_barrier()
        pltpu.sync_copy(sh_out.at[pl.ds(base, per_w)], out_hbm.at[b, pl.ds(base, per_w)])

    return k(codes)


def _attn_body(rows_ref, fc2_b, out_ref):
    K = rows_ref.shape[1] // _CHUNK
    eye = jnp.eye(_CHUNK, dtype=jnp.float32)
    zpad = jnp.zeros((_CHUNK, _OROW - _CHANNELS - 1), dtype=jnp.float32)

    def chunk(start):
        return rows_ref[0, pl.ds(start, _CHUNK), :]

    t0 = _CR + _CHANNELS

    def body(k, _):
        cur = chunk(k * _CHUNK)
        prv = chunk(lax.rem(k + K - 1, K) * _CHUNK)
        nxt = chunk(lax.rem(k + 1, K) * _CHUNK)
        xq = cur[:, :_CR]

        def nrm(t):
            x = t[:, :_CR]
            n = jnp.sqrt(jnp.sum(x * x, axis=-1, keepdims=True))
            return x / jnp.maximum(n, 5e-05)

        xm = jnp.concatenate([nrm(cur), nrm(prv), nrm(nxt)], axis=0)
        yc = jnp.concatenate([cur[:, _CR:t0],
                              prv[:, _CR:t0],
                              nxt[:, _CR:t0]], axis=0)
        tc = jnp.concatenate([cur[:, t0:t0 + _CHUNK],
                              prv[:, t0:t0 + _CHUNK],
                              nxt[:, t0:t0 + _CHUNK]], axis=0)

        raw_t = lax.dot_general(xm, xq, (((1,), (1,)), ((), ())),
                                preferred_element_type=jnp.float32) + tc + fc2_b[...]
        m = jnp.max(raw_t, axis=0, keepdims=True)
        e = jnp.exp(raw_t - m)
        s = jnp.sum(e, axis=0, keepdims=True)
        ret = lax.dot_general(e, yc, (((0,), (0,)), ((), ())),
                              preferred_element_type=jnp.float32)
        bsms = m + jnp.log(s)
        bscol = lax.dot_general(eye, bsms, (((1,), (1,)), ((), ())),
                                preferred_element_type=jnp.float32)
        scol = lax.dot_general(eye, s, (((1,), (1,)), ((), ())),
                               preferred_element_type=jnp.float32)
        ret = ret * (1.0 / scol)
        out_ref[0, pl.ds(k * _CHUNK, _CHUNK), :] = jnp.concatenate(
            [ret, bscol, zpad], axis=1)
        return 0

    lax.fori_loop(0, K, body, 0)


def _bucket_attention(rows_sorted, fc2_b, G, LH):
    return pl.pallas_call(
        _attn_body,
        grid=(G,),
        in_specs=[
            pl.BlockSpec((1, LH, _ROW), lambda h: (h, 0, 0)),
            pl.BlockSpec((1, _CHUNK), lambda h: (0, 0)),
        ],
        out_specs=pl.BlockSpec((1, LH, _OROW), lambda h: (h, 0, 0)),
        out_shape=jax.ShapeDtypeStruct((G, LH, _OROW), jnp.float32),
    )(rows_sorted, fc2_b.reshape(1, -1))


def kernel(input, cm_w, cm_b, ca_w, ca_b, cf_w, cf_b, fc1_w, fc1_b, fc2_w, fc2_b, random_rotations):
    N, H, W, _ = input.shape
    L = H * W

    table, codes = _front(input, cm_w, cm_b, ca_w, ca_b, cf_w, cf_b,
                          fc1_w, fc1_b, fc2_w, random_rotations)

    hash_codes = codes.transpose(0, 2, 1).reshape(N, -1)
    indices = _sc_counting_sort(hash_codes.astype(jnp.int32), N, hash_codes.shape[1])

    HL = _N_HASHES * L
    gidx = (indices % L + (jnp.arange(N) * L)[:, None]).astype(jnp.int32)
    rows_sorted = _sc_gather(table, gidx.reshape(_NW, -1, _SCCHUNK), _ROW)

    G = N * _N_HASHES
    LH = L
    out80 = _bucket_attention(rows_sorted.reshape(G, LH, _ROW), fc2_b, G, LH)

    gdst = (indices + (jnp.arange(N) * HL)[:, None]).astype(jnp.int32)
    unsorted = _sc_scatter(out80.reshape(N * HL, _OROW), gdst.reshape(_NW, -1, _SCCHUNK), _OROW)

    ret = unsorted[:, :_CHANNELS].reshape(N, _N_HASHES, L, _CHANNELS)
    bscore = unsorted[:, _CHANNELS].reshape(N, _N_HASHES, L, 1)
    probs = jax.nn.softmax(bscore, axis=1)
    ret = jnp.sum(ret * probs, axis=1).reshape(N, H, W, -1)
    return ret + input
